# baseline (speedup 1.0000x reference)
import functools
import math

import jax
import jax.numpy as jnp
from jax import lax
from jax.experimental import pallas as pl
from jax.experimental.pallas import tpu as pltpu
from jax.experimental.pallas import tpu_sc as plsc

_D = 128
_SCALE = math.sqrt(_D)

_NC = 2
_NS = 16
_NW = _NC * _NS

_CHUNK = 200
_NBUF = 4
_AHEAD = 2
_L = 16


def _make_gather(n_rows):
    b_per_w = n_rows // _NW
    n_chunks = b_per_w // _CHUNK
    assert n_rows % (_NW * _CHUNK) == 0 and n_chunks % _NBUF == 0, n_rows
    mesh = plsc.VectorSubcoreMesh(core_axis_name="c", subcore_axis_name="s")

    @functools.partial(
        pl.kernel,
        out_type=jax.ShapeDtypeStruct((n_rows // _CHUNK, _CHUNK, _D),
                                      jnp.float32),
        mesh=mesh,
        scratch_types=[
            pltpu.VMEM((b_per_w,), jnp.int32),
            pltpu.VMEM((_NBUF, _CHUNK, _D), jnp.float32),
            [pltpu.SemaphoreType.DMA] * _NBUF,
            [pltpu.SemaphoreType.DMA] * (_NBUF // 2),
        ],
    )
    def gather(table_hbm, idx_hbm, out_hbm, idx_v, rows_v, gsems, wsems):
        wid = lax.axis_index("s") * _NC + lax.axis_index("c")
        blk0 = wid * n_chunks

        pltpu.sync_copy(idx_hbm.at[wid], idx_v)

        def fire(chunk, slot):
            idx = idx_v.at[pl.ds(chunk * _CHUNK, _CHUNK)]
            pltpu.async_copy(table_hbm.at[idx], rows_v.at[slot], gsems[slot])

        def wb_pair(even_chunk, even_slot):
            return pltpu.make_async_copy(
                rows_v.at[pl.ds(even_slot, 2)],
                out_hbm.at[pl.ds(blk0 + even_chunk, 2)],
                wsems[even_slot // 2])

        def drain(chunk, slot):
            idx = idx_v.at[pl.ds(chunk * _CHUNK, _CHUNK)]
            pltpu.make_async_copy(table_hbm.at[idx], rows_v.at[slot],
                                  gsems[slot]).wait()

            def scale_row(r, _):
                for j in range(_D // _L):
                    sl = pl.ds(j * _L, _L)
                    rows_v[slot, r, sl] = rows_v[slot, r, sl] * _SCALE
                return ()

            lax.fori_loop(0, _CHUNK, scale_row, (), unroll=2)
            if slot % 2 == 1:
                wb_pair(chunk - 1, slot - 1).start()

        for b in range(_AHEAD):
            fire(b, b)

        def body(g, _):
            i = g * _NBUF
            for b in range(_NBUF):
                nxt = i + b + _AHEAD
                slot_n = (b + _AHEAD) % _NBUF

                @pl.when(nxt < n_chunks)
                def _():
                    if slot_n % 2 == 0:
                        @pl.when(nxt >= _NBUF)
                        def _():
                            wb_pair(nxt - _NBUF, slot_n).wait()

                    fire(nxt, slot_n)

                drain(i + b, b)
            return ()

        lax.fori_loop(0, n_chunks // _NBUF, body, ())

        for p in range(_NBUF // 2):
            wb_pair(n_chunks - _NBUF + 2 * p, 2 * p).wait()

    return gather


_gather = _make_gather(4096 * 200)


def kernel(x, lut):
    b, s = x.shape
    n = b * s
    idx = x.reshape(_NW, n // _NW).astype(jnp.int32)
    out = _gather(lut, idx)
    return out.reshape(b, s, _D)


# speedup vs baseline: 1.0033x; 1.0033x over previous
import functools
import math

import jax
import jax.numpy as jnp
from jax import lax
from jax.experimental import pallas as pl
from jax.experimental.pallas import tpu as pltpu
from jax.experimental.pallas import tpu_sc as plsc

_D = 128
_SCALE = math.sqrt(_D)

_NC = 2
_NS = 16
_NW = _NC * _NS

_CHUNK = 200
_NBUF = 4
_L = 16


def _make_gather(n_rows):
    b_per_w = n_rows // _NW
    n_chunks = b_per_w // _CHUNK
    assert n_rows % (_NW * _CHUNK) == 0 and n_chunks % _NBUF == 0, n_rows
    mesh = plsc.VectorSubcoreMesh(core_axis_name="c", subcore_axis_name="s")

    @functools.partial(
        pl.kernel,
        out_type=jax.ShapeDtypeStruct((n_rows, _D), jnp.float32),
        mesh=mesh,
        scratch_types=[
            pltpu.VMEM((b_per_w,), jnp.int32),
            pltpu.VMEM((_NBUF, _CHUNK, _D), jnp.float32),
            [pltpu.SemaphoreType.DMA] * _NBUF,
            [pltpu.SemaphoreType.DMA] * _NBUF,
        ],
    )
    def gather(table_hbm, idx_hbm, out_hbm, idx_v, rows_v, gsems, wsems):
        wid = lax.axis_index("s") * _NC + lax.axis_index("c")
        base = wid * b_per_w

        pltpu.sync_copy(idx_hbm.at[wid], idx_v)

        def fire(chunk, slot):
            idx = idx_v.at[pl.ds(chunk * _CHUNK, _CHUNK)]
            pltpu.async_copy(table_hbm.at[idx], rows_v.at[slot], gsems[slot])

        def wb_copy(chunk, slot):
            off = base + chunk * _CHUNK
            return pltpu.make_async_copy(
                rows_v.at[slot], out_hbm.at[pl.ds(off, _CHUNK)], wsems[slot])

        def drain(chunk, slot):
            idx = idx_v.at[pl.ds(chunk * _CHUNK, _CHUNK)]
            pltpu.make_async_copy(table_hbm.at[idx], rows_v.at[slot],
                                  gsems[slot]).wait()

            def scale_row(r, _):
                for j in range(_D // _L):
                    sl = pl.ds(j * _L, _L)
                    rows_v[slot, r, sl] = rows_v[slot, r, sl] * _SCALE
                return ()

            lax.fori_loop(0, _CHUNK, scale_row, (), unroll=2)
            wb_copy(chunk, slot).start()

        for b in range(_NBUF - 1):
            fire(b, b)

        def body(g, _):
            i = g * _NBUF
            for b in range(_NBUF):
                nxt = i + b + _NBUF - 1
                slot_n = (b + _NBUF - 1) % _NBUF

                @pl.when(nxt < n_chunks)
                def _():
                    @pl.when(nxt >= _NBUF)
                    def _():
                        wb_copy(nxt - _NBUF, slot_n).wait()

                    fire(nxt, slot_n)

                drain(i + b, b)
            return ()

        lax.fori_loop(0, n_chunks // _NBUF, body, ())

        for b in range(_NBUF):
            wb_copy(n_chunks - _NBUF + b, b).wait()

    return gather


_gather = _make_gather(4096 * 200)


def kernel(x, lut):
    b, s = x.shape
    n = b * s
    idx = x.reshape(_NW, n // _NW).astype(jnp.int32)
    out = _gather(lut, idx)
    return out.reshape(b, s, _D)
